# Initial kernel scaffold; baseline (speedup 1.0000x reference)
import jax
import jax.numpy as jnp
from jax.experimental import pallas as pl


def kernel(embeddings_t, target_t):
    raise NotImplementedError("write your pallas kernel here")


# speedup vs baseline: 665.3618x; 665.3618x over previous
import jax
import jax.numpy as jnp
from jax.experimental import pallas as pl

MARGIN = 1.0
BLK = 128


def _loss_body(x_ref, xt_ref, lc_ref, lr_ref, out_ref):
    i = pl.program_id(0)

    @pl.when(i == 0)
    def _init():
        out_ref[...] = jnp.zeros_like(out_ref)

    x = x_ref[...]
    xt = xt_ref[...]
    g = jax.lax.dot_general(
        x, xt, (((1,), (0,)), ((), ())),
        preferred_element_type=jnp.float32)
    n_col = jnp.sum(x * x, axis=1, keepdims=True)
    n_row = jnp.sum(xt * xt, axis=0, keepdims=True)
    d = jnp.maximum(n_col + n_row - 2.0 * g, 0.0)
    eq = lc_ref[...] == lr_ref[...]
    neg = jnp.maximum(MARGIN - jnp.sqrt(d), 0.0)
    loss = jnp.where(eq, d, neg * neg)
    n_total = xt.shape[1]
    scale = 1.0 / (n_total * (n_total - 1.0))
    out_ref[...] += jnp.sum(loss, keepdims=True) * scale


def kernel(embeddings_t, target_t):
    n, d = embeddings_t.shape
    xt = embeddings_t.T
    lc = target_t.reshape(n, 1)
    lr = target_t.reshape(1, n)
    out = pl.pallas_call(
        _loss_body,
        grid=(n // BLK,),
        in_specs=[
            pl.BlockSpec((BLK, d), lambda i: (i, 0)),
            pl.BlockSpec((d, n), lambda i: (0, 0)),
            pl.BlockSpec((BLK, 1), lambda i: (i, 0)),
            pl.BlockSpec((1, n), lambda i: (0, 0)),
        ],
        out_specs=pl.BlockSpec((1, 1), lambda i: (0, 0)),
        out_shape=jax.ShapeDtypeStruct((1, 1), jnp.float32),
    )(embeddings_t, xt, lc, lr)
    return out[0, 0]


# speedup vs baseline: 760.0552x; 1.1423x over previous
import jax
import jax.numpy as jnp
from jax.experimental import pallas as pl
from jax.experimental.pallas import tpu as pltpu

MARGIN = 1.0
BLK = 128


def _loss_body(x_ref, lc_ref, lr_ref, out_ref, nrow_ref):
    i = pl.program_id(0)
    x_all = x_ref[...]

    @pl.when(i == 0)
    def _init():
        out_ref[...] = jnp.zeros_like(out_ref)
        sq = x_all * x_all
        ones = jnp.ones((1, x_all.shape[1]), jnp.float32)
        nrow_ref[...] = jax.lax.dot_general(
            ones, sq, (((1,), (1,)), ((), ())),
            preferred_element_type=jnp.float32)

    x = x_ref[pl.ds(i * BLK, BLK), :]
    g = jax.lax.dot_general(
        x, x_all, (((1,), (1,)), ((), ())),
        preferred_element_type=jnp.float32)
    n_col = jnp.sum(x * x, axis=1, keepdims=True)
    n_row = nrow_ref[...]
    d = jnp.maximum(n_col + n_row - 2.0 * g, 0.0)
    eq = lc_ref[...] == lr_ref[...]
    neg = jnp.maximum(MARGIN - jnp.sqrt(d), 0.0)
    loss = jnp.where(eq, d, neg * neg)
    n_total = x_all.shape[0]
    scale = 1.0 / (n_total * (n_total - 1.0))
    out_ref[...] += jnp.sum(loss, keepdims=True) * scale


def kernel(embeddings_t, target_t):
    n, d = embeddings_t.shape
    lc = target_t.reshape(n, 1)
    lr = target_t.reshape(1, n)
    out = pl.pallas_call(
        _loss_body,
        grid=(n // BLK,),
        in_specs=[
            pl.BlockSpec((n, d), lambda i: (0, 0)),
            pl.BlockSpec((BLK, 1), lambda i: (i, 0)),
            pl.BlockSpec((1, n), lambda i: (0, 0)),
        ],
        out_specs=pl.BlockSpec((1, 1), lambda i: (0, 0)),
        out_shape=jax.ShapeDtypeStruct((1, 1), jnp.float32),
        scratch_shapes=[pltpu.VMEM((1, n), jnp.float32)],
    )(embeddings_t, lc, lr)
    return out[0, 0]


# speedup vs baseline: 1047.0631x; 1.3776x over previous
import jax
import jax.numpy as jnp
from jax.experimental import pallas as pl

MARGIN = 1.0


def _loss_body(x_ref, lc_ref, lr_ref, out_ref):
    x = x_ref[...]
    g = jax.lax.dot_general(
        x, x, (((1,), (1,)), ((), ())),
        preferred_element_type=jnp.float32)
    n_col = jnp.sum(x * x, axis=1, keepdims=True)
    n_row = jax.lax.dot_general(
        jnp.ones((1, x.shape[1]), jnp.float32), x * x,
        (((1,), (1,)), ((), ())),
        preferred_element_type=jnp.float32)
    d = jnp.maximum(n_col + n_row - 2.0 * g, 0.0)
    eq = lc_ref[...] == lr_ref[...]
    neg = jnp.maximum(MARGIN - jnp.sqrt(d), 0.0)
    loss = jnp.where(eq, d, neg * neg)
    n_total = x.shape[0]
    scale = 1.0 / (n_total * (n_total - 1.0))
    out_ref[...] = jnp.sum(loss, keepdims=True) * scale


def kernel(embeddings_t, target_t):
    n, _ = embeddings_t.shape
    lc = target_t.reshape(n, 1)
    lr = target_t.reshape(1, n)
    out = pl.pallas_call(
        _loss_body,
        out_shape=jax.ShapeDtypeStruct((1, 1), jnp.float32),
    )(embeddings_t, lc, lr)
    return out[0, 0]


# speedup vs baseline: 1571.9685x; 1.5013x over previous
import jax
import jax.numpy as jnp
from jax.experimental import pallas as pl

MARGIN = 1.0
BLK = 128


def _loss_body(x_ref, lc_ref, lr_ref, out_ref):
    x = x_ref[...]
    n_total, dim = x.shape
    nb = n_total // BLK
    nrow = jax.lax.dot_general(
        jnp.ones((1, dim), jnp.float32), x * x,
        (((1,), (1,)), ((), ())),
        preferred_element_type=jnp.float32)

    acc_diag = jnp.zeros((BLK, BLK), jnp.float32)
    acc_off = jnp.zeros((BLK, BLK), jnp.float32)
    for i in range(nb):
        xi = x_ref[pl.ds(i * BLK, BLK), :]
        ni = jnp.sum(xi * xi, axis=1, keepdims=True)
        li = lc_ref[pl.ds(i * BLK, BLK), :]
        for j in range(i, nb):
            xj = x_ref[pl.ds(j * BLK, BLK), :]
            g = jax.lax.dot_general(
                xi, xj, (((1,), (1,)), ((), ())),
                preferred_element_type=jnp.float32)
            nj = nrow[:, j * BLK:(j + 1) * BLK]
            d = jnp.maximum(ni + nj - 2.0 * g, 0.0)
            eq = li == lr_ref[:, pl.ds(j * BLK, BLK)]
            sqrt_d = d * jax.lax.rsqrt(d + 1e-37)
            neg = jnp.maximum(MARGIN - sqrt_d, 0.0)
            loss = jnp.where(eq, d, neg * neg)
            if i == j:
                acc_diag = acc_diag + loss
            else:
                acc_off = acc_off + loss
    total = jnp.sum(acc_diag) + 2.0 * jnp.sum(acc_off)
    scale = 1.0 / (n_total * (n_total - 1.0))
    out_ref[...] = jnp.full((1, 1), scale) * total


def kernel(embeddings_t, target_t):
    n, _ = embeddings_t.shape
    lc = target_t.reshape(n, 1)
    lr = target_t.reshape(1, n)
    out = pl.pallas_call(
        _loss_body,
        out_shape=jax.ShapeDtypeStruct((1, 1), jnp.float32),
    )(embeddings_t, lc, lr)
    return out[0, 0]
